# baseline (speedup 1.0000x reference)
import functools

import jax
import jax.numpy as jnp
from jax import lax
from jax.experimental import pallas as pl
from jax.experimental.pallas import tpu as pltpu
from jax.experimental.pallas import tpu_sc as plsc

NUM_CORES = 2
NUM_SUBCORES = 16
NW = NUM_CORES * NUM_SUBCORES
CH = 64
SUB = 8


def _window(emb, mem, q, r, bp):
    b = emb.shape[0]
    m = mem.shape[0]
    w = jnp.zeros((bp,) + emb.shape[1:], emb.dtype)
    w = lax.dynamic_update_slice(w, emb, (r,) + (0,) * (emb.ndim - 1))
    i8 = jnp.arange(SUB).reshape((SUB,) + (1,) * (emb.ndim - 1))
    head = lax.dynamic_slice_in_dim(mem, q, SUB, axis=0)
    w = w.at[0:SUB].set(jnp.where(i8 < r, head, w[0:SUB]))
    tail = mem[(q + b + jnp.arange(SUB)) % m]
    w = w.at[b:bp].set(jnp.where(i8 >= r, tail, w[b:bp]))
    return w


def kernel(mem_img, mem_gps, mem_coords, img_emb, gps_emb, gps_coords, ptr):
    M, D = mem_img.shape
    B = img_emb.shape[0]
    C = mem_coords.shape[1]
    W = 2 * D + C
    CP = 128
    BP = B + SUB
    rows_per_w = M // NW
    n_chunks = rows_per_w // CH

    p = jnp.asarray(ptr, jnp.int32) % jnp.int32(M)
    q = p & jnp.int32(-SUB)
    r = p & jnp.int32(SUB - 1)
    q_vec = jnp.full((16,), q, dtype=jnp.int32)

    ie2 = _window(img_emb, mem_img, q, r, BP)
    ge2 = _window(gps_emb, mem_gps, q, r, BP)
    gc2 = jnp.pad(_window(gps_coords, mem_coords, q, r, BP),
                  ((0, 0), (0, CP - C)))
    mc_u = jnp.pad(mem_coords, ((0, 0), (0, CP - C)))

    mesh = plsc.VectorSubcoreMesh(core_axis_name="c", subcore_axis_name="s")

    @functools.partial(
        pl.kernel,
        out_type=(jax.ShapeDtypeStruct((M, W), jnp.float32),
                  jax.ShapeDtypeStruct((M, CP), jnp.float32)),
        mesh=mesh,
        scratch_types=[
            pltpu.VMEM((CH, D), jnp.float32),
            pltpu.VMEM((CH, D), jnp.float32),
            pltpu.VMEM((CH, CP), jnp.float32),
            pltpu.VMEM((16,), jnp.int32),
        ],
    )
    def run(mi, mg, mc, ie, ge, gc, qv, out, ocrd, bimg, bgps, bcrd, qbuf):
        wid = lax.axis_index("s") * NUM_CORES + lax.axis_index("c")
        base = wid * rows_per_w
        pltpu.sync_copy(qv, qbuf)
        qk = qbuf[...][0]

        def chunk_body(t, carry):
            c0 = pl.multiple_of(base + t * CH, CH)
            pltpu.sync_copy(mi.at[pl.ds(c0, CH)], bimg)
            pltpu.sync_copy(mg.at[pl.ds(c0, CH)], bgps)
            pltpu.sync_copy(mc.at[pl.ds(c0, CH)], bcrd)

            d = c0 - qk
            s = jnp.where(d < 0, d + M, d)
            for k in range(CH // SUB):
                e = s + k * SUB
                e = jnp.where(e >= M, e - M, e)

                @pl.when(e < BP)
                def _overlay(e=e, k=k):
                    ea = pl.multiple_of(e, SUB)
                    pltpu.sync_copy(ie.at[pl.ds(ea, SUB)],
                                    bimg.at[pl.ds(k * SUB, SUB)])
                    pltpu.sync_copy(ge.at[pl.ds(ea, SUB)],
                                    bgps.at[pl.ds(k * SUB, SUB)])
                    pltpu.sync_copy(gc.at[pl.ds(ea, SUB)],
                                    bcrd.at[pl.ds(k * SUB, SUB)])

            pltpu.sync_copy(bimg, out.at[pl.ds(c0, CH), pl.ds(0, D)])
            pltpu.sync_copy(bgps, out.at[pl.ds(c0, CH), pl.ds(D, D)])
            pltpu.sync_copy(bcrd, ocrd.at[pl.ds(c0, CH)])
            return carry

        lax.fori_loop(0, n_chunks, chunk_body, 0)

    out_sc, out_crd = run(mem_img, mem_gps, mc_u, ie2, ge2, gc2, q_vec)

    BRT = 512

    def tc_body(_, crd_ref, o_ref):
        o_ref[...] = crd_ref[...]

    return pl.pallas_call(
        tc_body,
        grid=(M // BRT,),
        in_specs=[
            pl.BlockSpec(memory_space=pl.ANY),
            pl.BlockSpec((BRT, CP), lambda i: (i, 0)),
        ],
        out_specs=pl.BlockSpec((BRT, CP), lambda i: (i, 2 * D // CP)),
        out_shape=jax.ShapeDtypeStruct((M, W), jnp.float32),
        input_output_aliases={0: 0},
    )(out_sc, out_crd)
